# Initial kernel scaffold; baseline (speedup 1.0000x reference)
import jax
import jax.numpy as jnp
from jax.experimental import pallas as pl


def kernel(x, edge_index, W1, att_src1, att_dst1, b1, W2, att_src2, att_dst2, b2):
    raise NotImplementedError("write your pallas kernel here")


# speedup vs baseline: 1.1284x; 1.1284x over previous
import functools

import jax
import jax.numpy as jnp
from jax.experimental import pallas as pl

N_NODES = 10000
HEADS = 8
HIDDEN = 256


def _mm_kernel(x_ref, w_ref, o_ref):
    o_ref[...] = jnp.dot(x_ref[...], w_ref[...],
                         preferred_element_type=jnp.float32)


def _matmul(x, w):
    m, k = x.shape
    k2, n = w.shape
    bm = 512
    grid = (m // bm,)
    return pl.pallas_call(
        _mm_kernel,
        grid=grid,
        in_specs=[pl.BlockSpec((bm, k), lambda i: (i, 0)),
                  pl.BlockSpec((k, n), lambda i: (0, 0))],
        out_specs=pl.BlockSpec((bm, n), lambda i: (i, 0)),
        out_shape=jax.ShapeDtypeStruct((m, n), jnp.float32),
    )(x, w)


def _gat_layer(x, edge_index, W, att_src, att_dst, bias, heads, out_ch, concat):
    n = x.shape[0]
    ar = jnp.arange(n, dtype=edge_index.dtype)
    src = jnp.concatenate([edge_index[0], ar])
    dst = jnp.concatenate([edge_index[1], ar])
    xp = jnp.pad(x, ((0, 0), (0, (-x.shape[1]) % 8)))
    wp = jnp.pad(W, ((0, (-W.shape[0]) % 8), (0, 0)))
    h = _matmul(xp, wp).reshape(n, heads, out_ch)
    a_src = jnp.sum(h * att_src, axis=-1)
    a_dst = jnp.sum(h * att_dst, axis=-1)
    alpha = a_src[src] + a_dst[dst]
    alpha = jax.nn.leaky_relu(alpha, 0.2)
    s = jnp.exp(alpha)
    denom = jax.ops.segment_sum(s, dst, num_segments=n)
    msg = h[src] * s[:, :, None]
    agg = jax.ops.segment_sum(msg, dst, num_segments=n)
    out = agg / (denom + 1e-16)[:, :, None]
    if concat:
        out = out.reshape(n, heads * out_ch)
    else:
        out = out.mean(axis=1)
    return out + bias


def kernel(x, edge_index, W1, att_src1, att_dst1, b1, W2, att_src2, att_dst2, b2):
    h = _gat_layer(x, edge_index, W1, att_src1, att_dst1, b1, HEADS, HIDDEN, True)
    h = jax.nn.elu(h)
    return _gat_layer(h, edge_index, W2, att_src2, att_dst2, b2, 1, HIDDEN, False)


# speedup vs baseline: 1.1330x; 1.0041x over previous
import jax
import jax.numpy as jnp
from jax.experimental import pallas as pl

N = 10000
E = 160000
HEADS = 8
HID = 256
F1 = HEADS * HID

BM = 1000


def _stage_a_body(x_ref, w_ref, a_ref, h_ref, ab_ref):
    h = jnp.dot(x_ref[...], w_ref[...], preferred_element_type=jnp.float32)
    ab = jnp.dot(h, a_ref[...], preferred_element_type=jnp.float32)
    h_ref[...] = h
    ab_ref[...] = ab


def _stage_a(xp, w1p, attcat):
    return pl.pallas_call(
        _stage_a_body,
        grid=(N // BM,),
        in_specs=[pl.BlockSpec((BM, 8), lambda i: (i, 0)),
                  pl.BlockSpec((8, F1), lambda i: (0, 0)),
                  pl.BlockSpec((F1, 16), lambda i: (0, 0))],
        out_specs=[pl.BlockSpec((BM, F1), lambda i: (i, 0)),
                   pl.BlockSpec((BM, 16), lambda i: (i, 0))],
        out_shape=[jax.ShapeDtypeStruct((N, F1), jnp.float32),
                   jax.ShapeDtypeStruct((N, 16), jnp.float32)],
    )(xp, w1p, attcat)


def _stage_c_body(agg_ref, den_ref, b_ref, w2_ref, a2_ref, h2_ref, ab2_ref):
    agg = agg_ref[...]
    den = den_ref[...]
    parts = [agg[:, hd * HID:(hd + 1) * HID] / (den[:, hd:hd + 1] + 1e-16)
             for hd in range(HEADS)]
    o = jnp.concatenate(parts, axis=1) + b_ref[...]
    o = jnp.where(o > 0, o, jnp.exp(jnp.minimum(o, 0.0)) - 1.0)
    h2 = jnp.dot(o, w2_ref[...], preferred_element_type=jnp.float32)
    ab2 = jnp.dot(h2, a2_ref[...], preferred_element_type=jnp.float32)
    h2_ref[...] = h2
    ab2_ref[...] = ab2


def _stage_c(agg1, den1, b1_2d, W2, attcat2):
    return pl.pallas_call(
        _stage_c_body,
        grid=(N // BM,),
        in_specs=[pl.BlockSpec((BM, F1), lambda i: (i, 0)),
                  pl.BlockSpec((BM, 16), lambda i: (i, 0)),
                  pl.BlockSpec((1, F1), lambda i: (0, 0)),
                  pl.BlockSpec((F1, HID), lambda i: (0, 0)),
                  pl.BlockSpec((HID, 16), lambda i: (0, 0))],
        out_specs=[pl.BlockSpec((BM, HID), lambda i: (i, 0)),
                   pl.BlockSpec((BM, 16), lambda i: (i, 0))],
        out_shape=[jax.ShapeDtypeStruct((N, HID), jnp.float32),
                   jax.ShapeDtypeStruct((N, 16), jnp.float32)],
    )(agg1, den1, b1_2d, W2, attcat2)


def _stage_e_body(agg_ref, den_ref, b_ref, o_ref):
    o_ref[...] = agg_ref[...] / (den_ref[:, 0:1] + 1e-16) + b_ref[...]


def _stage_e(agg2, den2, b2_2d):
    return pl.pallas_call(
        _stage_e_body,
        grid=(N // BM,),
        in_specs=[pl.BlockSpec((BM, HID), lambda i: (i, 0)),
                  pl.BlockSpec((BM, 16), lambda i: (i, 0)),
                  pl.BlockSpec((1, HID), lambda i: (0, 0))],
        out_specs=pl.BlockSpec((BM, HID), lambda i: (i, 0)),
        out_shape=jax.ShapeDtypeStruct((N, HID), jnp.float32),
    )(agg2, den2, b2_2d)


def _attcat_from(att_src, att_dst, heads, hid):
    acat = jnp.zeros((heads * hid, 16), jnp.float32)
    for hd in range(heads):
        acat = acat.at[hd * hid:(hd + 1) * hid, hd].set(att_src[0, hd])
        acat = acat.at[hd * hid:(hd + 1) * hid, 8 + hd].set(att_dst[0, hd])
    return acat


def _edge_agg(h, ab, src, dst, heads):
    a_src = ab[:, 0:heads]
    a_dst = ab[:, 8:8 + heads]
    al = a_src[src] + a_dst[dst]
    al = jnp.where(al >= 0, al, 0.2 * al)
    s = jnp.exp(al)
    den = jax.ops.segment_sum(s, dst, num_segments=N)
    hh = h.reshape(N, heads, -1)
    msg = hh[src] * s[:, :, None]
    agg = jax.ops.segment_sum(msg, dst, num_segments=N).reshape(N, -1)
    den16 = jnp.concatenate(
        [den, jnp.zeros((N, 16 - heads), jnp.float32)], axis=1)
    return agg, den16


def kernel(x, edge_index, W1, att_src1, att_dst1, b1, W2, att_src2, att_dst2,
           b2):
    xp = jnp.pad(x, ((0, 0), (0, 1)))
    w1p = jnp.pad(W1, ((0, 1), (0, 0)))
    attcat1 = _attcat_from(att_src1, att_dst1, HEADS, HID)
    attcat2 = _attcat_from(att_src2, att_dst2, 1, HID)
    ar = jnp.arange(N, dtype=edge_index.dtype)
    src = jnp.concatenate([edge_index[0], ar])
    dst = jnp.concatenate([edge_index[1], ar])

    h1, ab1 = _stage_a(xp, w1p, attcat1)
    agg1, den1 = _edge_agg(h1, ab1, src, dst, HEADS)
    h2, ab2 = _stage_c(agg1, den1, b1.reshape(1, F1), W2, attcat2)
    agg2, den2 = _edge_agg(h2, ab2, src, dst, 1)
    return _stage_e(agg2, den2, b2.reshape(1, HID))
